# baseline (speedup 1.0000x reference)
import functools

import jax
import jax.numpy as jnp
from jax import lax
from jax.experimental import pallas as pl
from jax.experimental.pallas import tpu as pltpu
from jax.experimental.pallas import tpu_sc as plsc

_WINDOW = 12
_NUM_HEAD = 16
_NUM_REL_DIST = (2 * _WINDOW - 1) ** 3
_N = 1024
_NROWS = _NUM_HEAD * _N
_SPAN = 19
_TBL = 2 * _SPAN - 1
_TBL_PAD = 40
_CHUNK = 128
_NCHUNKS = _SPAN * _N // _CHUNK
_DMA_BATCH = 16
_GATHER_BATCH = 8


def _s_val(i):
    i32 = jnp.int32
    a = lax.div(i, i32(144))
    b_lo = lax.rem(lax.div(i + i32(864), i32(12)), i32(12))
    b_hi = lax.rem(i - i32(864), i32(12))
    m = lax.min(lax.max(i32(864) - i, i32(0)), i32(1))
    return a + m * b_lo + (i32(1) - m) * b_hi


def _sc_body(tt_hbm, out_hbm, th_s, th_v, idx_v, g_v, u2_v, sem):
    cid = lax.axis_index("c")
    sid = lax.axis_index("s")
    pltpu.sync_copy(tt_hbm.at[pl.ds(sid * _TBL_PAD, _TBL_PAD)], th_v)
    pltpu.sync_copy(th_v, th_s.at[pl.ds(sid * _TBL_PAD, _TBL_PAD)])

    off = sid * _TBL_PAD + (_SPAN - 1)

    def build_idx(j16, _):
        base = j16 * 16
        j = base + lax.broadcasted_iota(jnp.int32, (16,), 0)
        d = off - _s_val(j)
        for v in range(_SPAN):
            idx_v[pl.ds(v * _N + base, 16)] = d + v
        return 0

    lax.fori_loop(0, _N // 16, build_idx, 0)

    def gather(g, _):
        descs = []
        for k in range(_GATHER_BATCH):
            o = (g * _GATHER_BATCH + k) * _CHUNK
            descs.append(pltpu.async_copy(
                th_s.at[idx_v.at[pl.ds(o, _CHUNK)]],
                g_v.at[pl.ds(o, _CHUNK)], sem))
        for d in descs:
            d.wait()
        return 0

    lax.fori_loop(0, _NCHUNKS // _GATHER_BATCH, gather, 0)

    for v in range(_SPAN):
        def repack(c, _, v=v):
            base = c * 16
            u2_v[v, pl.ds(base, 16)] = g_v[pl.ds(v * _N + base, 16)]
            return 0
        lax.fori_loop(0, _N // 16, repack, 0)

    i0 = cid * (_N // 2)

    def emit(b, _):
        descs = []
        for k in range(_DMA_BATCH):
            i = i0 + b * _DMA_BATCH + k
            v = _s_val(i)
            descs.append(pltpu.async_copy(
                u2_v.at[pl.ds(v, 1)],
                out_hbm.at[pl.ds(sid * _N + i, 1)], sem))
        for d in descs:
            d.wait()
        return 0

    lax.fori_loop(0, (_N // 2) // _DMA_BATCH, emit, 0)


@jax.jit
def _run(tt):
    mesh = plsc.VectorSubcoreMesh(core_axis_name="c", subcore_axis_name="s")
    f = functools.partial(
        pl.kernel,
        out_type=jax.ShapeDtypeStruct((_NROWS, _N), jnp.float32),
        mesh=mesh,
        scratch_types=[
            pltpu.VMEM_SHARED((_NUM_HEAD * _TBL_PAD,), jnp.float32),
            pltpu.VMEM((_TBL_PAD,), jnp.float32),
            pltpu.VMEM((_SPAN * _N,), jnp.int32),
            pltpu.VMEM((_SPAN * _N,), jnp.float32),
            pltpu.VMEM((_SPAN, _N), jnp.float32),
            pltpu.SemaphoreType.DMA,
        ],
    )(_sc_body)
    return f(tt)


def kernel(n, rpbt):
    del n
    t = jnp.concatenate(
        [rpbt[_NUM_REL_DIST - (_SPAN - 1):], rpbt[:_SPAN]], axis=0)
    tt = jnp.pad(jnp.transpose(t), ((0, 0), (0, _TBL_PAD - _TBL)))
    out = _run(jnp.reshape(tt, (-1,)))
    return jnp.reshape(out, (_NUM_HEAD, _N, _N))


# speedup vs baseline: 1.0549x; 1.0549x over previous
import functools

import jax
import jax.numpy as jnp
from jax import lax
from jax.experimental import pallas as pl
from jax.experimental.pallas import tpu as pltpu
from jax.experimental.pallas import tpu_sc as plsc

_WINDOW = 12
_NUM_HEAD = 16
_NUM_REL_DIST = (2 * _WINDOW - 1) ** 3
_N = 1024
_NROWS = _NUM_HEAD * _N
_SPAN = 19
_TBL = 2 * _SPAN - 1
_TBL_PAD = 40
_CHUNK = 128
_NCHUNKS = _SPAN * _N // _CHUNK
_DMA_BATCH = 32
_GATHER_BATCH = 8


def _s_val(i):
    i32 = jnp.int32
    a = lax.div(i, i32(144))
    b_lo = lax.rem(lax.div(i + i32(864), i32(12)), i32(12))
    b_hi = lax.rem(i - i32(864), i32(12))
    m = lax.min(lax.max(i32(864) - i, i32(0)), i32(1))
    return a + m * b_lo + (i32(1) - m) * b_hi


def _sc_body(tt_hbm, out_hbm, th_s, th_v, idx_v, g_v, u2_v, sem):
    cid = lax.axis_index("c")
    sid = lax.axis_index("s")
    pltpu.sync_copy(tt_hbm.at[pl.ds(sid * _TBL_PAD, _TBL_PAD)], th_v)
    pltpu.sync_copy(th_v, th_s.at[pl.ds(sid * _TBL_PAD, _TBL_PAD)])

    off = sid * _TBL_PAD + (_SPAN - 1)

    def build_idx(j16, _):
        base = j16 * 16
        j = base + lax.broadcasted_iota(jnp.int32, (16,), 0)
        d = off - _s_val(j)
        for v in range(_SPAN):
            idx_v[pl.ds(v * _N + base, 16)] = d + v
        return 0

    lax.fori_loop(0, _N // 16, build_idx, 0)

    def fire_row(v):
        descs = []
        for k in range(_N // _CHUNK):
            o = v * _N + k * _CHUNK
            descs.append(pltpu.async_copy(
                th_s.at[idx_v.at[pl.ds(o, _CHUNK)]],
                g_v.at[pl.ds(o, _CHUNK)], sem))
        return descs

    def repack_row(v):
        def repack(c, _):
            base = c * 16
            u2_v[v, pl.ds(base, 16)] = g_v[pl.ds(v * _N + base, 16)]
            return 0
        lax.fori_loop(0, _N // 16, repack, 0)

    for d in fire_row(0):
        d.wait()
    for v in range(1, _SPAN):
        descs = fire_row(v)
        repack_row(v - 1)
        for d in descs:
            d.wait()
    repack_row(_SPAN - 1)

    i0 = cid * (_N // 2)

    def emit(b, _):
        descs = []
        for k in range(_DMA_BATCH):
            i = i0 + b * _DMA_BATCH + k
            v = _s_val(i)
            descs.append(pltpu.async_copy(
                u2_v.at[pl.ds(v, 1)],
                out_hbm.at[pl.ds(sid * _N + i, 1)], sem))
        for d in descs:
            d.wait()
        return 0

    lax.fori_loop(0, (_N // 2) // _DMA_BATCH, emit, 0)


@jax.jit
def _run(tt):
    mesh = plsc.VectorSubcoreMesh(core_axis_name="c", subcore_axis_name="s")
    f = functools.partial(
        pl.kernel,
        out_type=jax.ShapeDtypeStruct((_NROWS, _N), jnp.float32),
        mesh=mesh,
        scratch_types=[
            pltpu.VMEM_SHARED((_NUM_HEAD * _TBL_PAD,), jnp.float32),
            pltpu.VMEM((_TBL_PAD,), jnp.float32),
            pltpu.VMEM((_SPAN * _N,), jnp.int32),
            pltpu.VMEM((_SPAN * _N,), jnp.float32),
            pltpu.VMEM((_SPAN, _N), jnp.float32),
            pltpu.SemaphoreType.DMA,
        ],
    )(_sc_body)
    return f(tt)


def kernel(n, rpbt):
    del n
    t = jnp.concatenate(
        [rpbt[_NUM_REL_DIST - (_SPAN - 1):], rpbt[:_SPAN]], axis=0)
    tt = jnp.pad(jnp.transpose(t), ((0, 0), (0, _TBL_PAD - _TBL)))
    out = _run(jnp.reshape(tt, (-1,)))
    return jnp.reshape(out, (_NUM_HEAD, _N, _N))


# speedup vs baseline: 1.0609x; 1.0057x over previous
import functools

import jax
import jax.numpy as jnp
from jax import lax
from jax.experimental import pallas as pl
from jax.experimental.pallas import tpu as pltpu
from jax.experimental.pallas import tpu_sc as plsc

_WINDOW = 12
_NUM_HEAD = 16
_NUM_REL_DIST = (2 * _WINDOW - 1) ** 3
_N = 1024
_NROWS = _NUM_HEAD * _N
_SPAN = 19
_TBL = 2 * _SPAN - 1
_TBL_PAD = 40
_CHUNK = 128
_DMA_BATCH = 32


def _s_val(i):
    i32 = jnp.int32
    a = lax.div(i, i32(144))
    b_lo = lax.rem(lax.div(i + i32(864), i32(12)), i32(12))
    b_hi = lax.rem(i - i32(864), i32(12))
    m = lax.min(lax.max(i32(864) - i, i32(0)), i32(1))
    return a + m * b_lo + (i32(1) - m) * b_hi


def _sc_body(tt_hbm, out_hbm, th_s, th_v, idx_v, g_v, u2_v, sem):
    cid = lax.axis_index("c")
    sid = lax.axis_index("s")
    pltpu.sync_copy(tt_hbm.at[pl.ds(sid * _TBL_PAD, _TBL_PAD)], th_v)
    pltpu.sync_copy(th_v, th_s.at[pl.ds(sid * _TBL_PAD, _TBL_PAD)])

    off = sid * _TBL_PAD + (_SPAN - 1)

    def build_idx(j16, _):
        base = j16 * 16
        j = base + lax.broadcasted_iota(jnp.int32, (16,), 0)
        d = off - _s_val(j)
        for v in range(_SPAN):
            idx_v[pl.ds(v * _N + base, 16)] = d + v
        return 0

    lax.fori_loop(0, _N // 16, build_idx, 0)

    def fire_row(v):
        descs = []
        for k in range(_N // _CHUNK):
            o = v * _N + k * _CHUNK
            descs.append(pltpu.async_copy(
                th_s.at[idx_v.at[pl.ds(o, _CHUNK)]],
                g_v.at[pl.ds(o, _CHUNK)], sem))
        return descs

    def repack_row(v):
        def repack(c, _):
            base = c * 16
            u2_v[v, pl.ds(base, 16)] = g_v[pl.ds(v * _N + base, 16)]
            return 0
        lax.fori_loop(0, _N // 16, repack, 0)

    for d in fire_row(0):
        d.wait()
    for v in range(1, _SPAN):
        descs = fire_row(v)
        repack_row(v - 1)
        for d in descs:
            d.wait()
    repack_row(_SPAN - 1)

    i0 = cid * (_N // 2)

    def emit(b, _):
        descs = []
        for k in range(_DMA_BATCH):
            i = i0 + b * _DMA_BATCH + k
            v = _s_val(i)
            descs.append(pltpu.async_copy(
                u2_v.at[pl.ds(v, 1)],
                out_hbm.at[pl.ds(sid * _N + i, 1)], sem))
        for d in descs:
            d.wait()
        return 0

    lax.fori_loop(0, (_N // 2) // _DMA_BATCH, emit, 0)


@jax.jit
def _run(tt):
    mesh = plsc.VectorSubcoreMesh(core_axis_name="c", subcore_axis_name="s")
    f = functools.partial(
        pl.kernel,
        out_type=jax.ShapeDtypeStruct((_NROWS, _N), jnp.float32),
        mesh=mesh,
        scratch_types=[
            pltpu.VMEM_SHARED((_NUM_HEAD * _TBL_PAD,), jnp.float32),
            pltpu.VMEM((_TBL_PAD,), jnp.float32),
            pltpu.VMEM((_SPAN * _N,), jnp.int32),
            pltpu.VMEM((_SPAN * _N,), jnp.float32),
            pltpu.VMEM((_SPAN, _N), jnp.float32),
            pltpu.SemaphoreType.DMA,
        ],
    )(_sc_body)
    return f(tt)


def kernel(n, rpbt):
    del n
    t = jnp.concatenate(
        [rpbt[_NUM_REL_DIST - (_SPAN - 1):], rpbt[:_SPAN]], axis=0)
    tt = jnp.pad(jnp.transpose(t), ((0, 0), (0, _TBL_PAD - _TBL)))
    out = _run(jnp.reshape(tt, (-1,)))
    return jnp.reshape(out, (_NUM_HEAD, _N, _N))
